# baseline (speedup 1.0000x reference)
import functools

import jax
import jax.numpy as jnp
from jax import lax
from jax.experimental import pallas as pl
from jax.experimental.pallas import tpu as pltpu
from jax.experimental.pallas import tpu_sc as plsc

_NC = 2
_NS = 16
_NW = _NC * _NS
_L = 16
_IDXW = 128


def _mf_dot_body(b_per_w, rows_per_w, emb_k,
                 uidx_hbm, iidx_hbm, w_hbm, h_hbm, out_hbm,
                 uidx_v, iidx_v, wrows_v, hrows_v, outc_v, sem):
    wid = lax.axis_index("s") * _NC + lax.axis_index("c")
    rbase = wid * rows_per_w

    pltpu.sync_copy(uidx_hbm.at[pl.ds(rbase, rows_per_w)], uidx_v)
    pltpu.sync_copy(iidx_hbm.at[pl.ds(rbase, rows_per_w)], iidx_v)

    copies = []
    for j in range(rows_per_w):
        copies.append(pltpu.async_copy(
            w_hbm.at[uidx_v.at[j]], wrows_v.at[pl.ds(j * _IDXW, _IDXW)], sem))
        copies.append(pltpu.async_copy(
            h_hbm.at[iidx_v.at[j]], hrows_v.at[pl.ds(j * _IDXW, _IDXW)], sem))
    for c in copies:
        c.wait()

    def g_body(g, carry):
        rows = g * _L + lax.iota(jnp.int32, _L)
        acc = jnp.zeros((_L,), jnp.float32)
        for k in range(emb_k):
            col = jnp.full((_L,), k, jnp.int32)
            wv = plsc.load_gather(wrows_v, [rows, col])
            hv = plsc.load_gather(hrows_v, [rows, col])
            acc = acc + wv * hv
        plsc.store_scatter(outc_v, [rows], acc)
        return carry

    lax.fori_loop(0, b_per_w // _L, g_body, 0)

    pltpu.sync_copy(outc_v, out_hbm.at[pl.ds(wid * b_per_w, b_per_w)])


def kernel(x, W, H, W_r, H_r):
    del W_r, H_r
    batch = x.shape[0]
    emb_k = W.shape[1]
    b_per_w = batch // _NW
    rows_per_w = b_per_w // _IDXW

    uidx = x[:, 0].reshape(batch // _IDXW, _IDXW)
    iidx = x[:, 1].reshape(batch // _IDXW, _IDXW)

    mesh = plsc.VectorSubcoreMesh(core_axis_name="c", subcore_axis_name="s")
    body = functools.partial(_mf_dot_body, b_per_w, rows_per_w, emb_k)
    fn = pl.kernel(
        body,
        out_type=jax.ShapeDtypeStruct((batch,), jnp.float32),
        mesh=mesh,
        scratch_types=[
            pltpu.VMEM((rows_per_w, _IDXW), jnp.int32),
            pltpu.VMEM((rows_per_w, _IDXW), jnp.int32),
            pltpu.VMEM((b_per_w, emb_k), jnp.float32),
            pltpu.VMEM((b_per_w, emb_k), jnp.float32),
            pltpu.VMEM((b_per_w,), jnp.float32),
            pltpu.SemaphoreType.DMA,
        ],
        compiler_params=pltpu.CompilerParams(
            needs_layout_passes=False, use_tc_tiling_on_sc=False),
    )
    return fn(uidx, iidx, W, H)
